# baseline (speedup 1.0000x reference)
import jax
import jax.numpy as jnp
from jax.experimental import pallas as pl
from jax.experimental.pallas import tpu as pltpu

_NSLAB = 8
_ROWS = 49152


def _dma_copy_body(x_hbm, o_hbm, v0, v1, bulk_sem, row_sem):
    r = _ROWS // _NSLAB
    bulk = [
        pltpu.make_async_copy(
            x_hbm.at[pl.ds(k * r, r), :], o_hbm.at[pl.ds(k * r, r), :], bulk_sem
        )
        for k in range(_NSLAB)
    ]
    for cp in bulk:
        cp.start()
    st0 = pltpu.make_async_copy(x_hbm.at[pl.ds(0, 1), :], v0, row_sem)
    st1 = pltpu.make_async_copy(x_hbm.at[pl.ds(16385, 1), :], v1, row_sem)
    st0.start()
    st1.start()
    st0.wait()
    st1.wait()
    v0[...] = v0[...] + jnp.float32(1.0)
    v1[...] = v1[...] + jnp.float32(1.0)
    for cp in bulk:
        cp.wait()
    wb0 = pltpu.make_async_copy(v0, o_hbm.at[pl.ds(0, 1), :], row_sem)
    wb1 = pltpu.make_async_copy(v1, o_hbm.at[pl.ds(16385, 1), :], row_sem)
    wb0.start()
    wb1.start()
    wb0.wait()
    wb1.wait()


def kernel(x):
    n, s, d = x.shape
    y = jnp.transpose(x, (1, 0, 2)).reshape(s * n, d)
    out = pl.pallas_call(
        _dma_copy_body,
        out_shape=jax.ShapeDtypeStruct((s * n, d), x.dtype),
        in_specs=[pl.BlockSpec(memory_space=pl.ANY)],
        out_specs=pl.BlockSpec(memory_space=pl.ANY),
        scratch_shapes=[
            pltpu.VMEM((1, d), jnp.float32),
            pltpu.VMEM((1, d), jnp.float32),
            pltpu.SemaphoreType.DMA,
            pltpu.SemaphoreType.DMA,
        ],
    )(y)
    return jnp.transpose(out.reshape(s, n, d), (1, 0, 2))


# speedup vs baseline: 48.4413x; 48.4413x over previous
import jax
import jax.numpy as jnp
from jax import lax
from jax.experimental import pallas as pl


def _scatter_add_body(x_ref, o_ref):
    i0 = lax.broadcasted_iota(jnp.int32, (2, 8, 1024), 0)
    i1 = lax.broadcasted_iota(jnp.int32, (2, 8, 1024), 1)
    hit = ((i0 == 0) & (i1 == 0)) | ((i0 == 1) & (i1 == 1))
    o_ref[...] = x_ref[...] + jnp.where(hit, jnp.float32(1.0), jnp.float32(0.0))


def kernel(x):
    n, s, d = x.shape
    xt = jnp.transpose(x, (1, 0, 2))
    out_t = pl.pallas_call(
        _scatter_add_body,
        out_shape=jax.ShapeDtypeStruct((s, n, d), x.dtype),
        grid=(1,),
        in_specs=[pl.BlockSpec((2, 8, d), lambda i: (0, 0, 0))],
        out_specs=pl.BlockSpec((2, 8, d), lambda i: (0, 0, 0)),
        input_output_aliases={0: 0},
    )(xt)
    return jnp.transpose(out_t, (1, 0, 2))


# speedup vs baseline: 49.1294x; 1.0142x over previous
import jax
import jax.numpy as jnp
from jax.experimental import pallas as pl

_BLK = 3072
_R0 = 0
_R1 = 16385


def _copy_scatter_body(x_ref, o_ref):
    i = pl.program_id(0)
    o_ref[...] = x_ref[...]

    @pl.when(i == _R0 // _BLK)
    def _():
        r = _R0 % _BLK
        o_ref[pl.ds(r, 1), :] = o_ref[pl.ds(r, 1), :] + jnp.float32(1.0)

    @pl.when(i == _R1 // _BLK)
    def _():
        r = _R1 % _BLK
        o_ref[pl.ds(r, 1), :] = o_ref[pl.ds(r, 1), :] + jnp.float32(1.0)


def kernel(x):
    n, s, d = x.shape
    y = jnp.transpose(x, (1, 0, 2)).reshape(s * n, d)
    out = pl.pallas_call(
        _copy_scatter_body,
        out_shape=jax.ShapeDtypeStruct((s * n, d), x.dtype),
        grid=(s * n // _BLK,),
        in_specs=[pl.BlockSpec((_BLK, d), lambda i: (i, 0))],
        out_specs=pl.BlockSpec((_BLK, d), lambda i: (i, 0)),
    )(y)
    return jnp.transpose(out.reshape(s, n, d), (1, 0, 2))
